# Initial kernel scaffold; baseline (speedup 1.0000x reference)
import jax
import jax.numpy as jnp
from jax.experimental import pallas as pl


def kernel(feat_center, feat_path, segment_ids, W_type, W_enc, W_dec, W_cls, b_cls):
    raise NotImplementedError("write your pallas kernel here")


# speedup vs baseline: 6.2013x; 6.2013x over previous
import functools
import math

import jax
import jax.numpy as jnp
from jax import lax
from jax.experimental import pallas as pl
from jax.experimental.pallas import tpu as pltpu
from jax.experimental.pallas import tpu_sc as plsc

_NC = 2
_NS = 16

_W1 = float(1.0 / (1.0 + math.exp(-0.5)))


def _sc_segment_sum(feat_path, ids, n_nodes):
    E, D = feat_path.shape
    NW = _NC * _NS
    rt = E // NW
    C = 80
    CSUB = 80
    KSUB = C // CSUB
    n_chunks = rt // C
    assert rt % C == 0 and C % 8 == 0 and n_chunks % 2 == 1
    zr = (n_nodes // _NS) // 8 * 8
    zl = n_nodes - zr * (_NS - 1)

    ids3d = ids.reshape(E // C, KSUB, CSUB)
    mesh = plsc.VectorSubcoreMesh(core_axis_name="c", subcore_axis_name="s")

    @functools.partial(
        pl.kernel,
        mesh=mesh,
        out_type=jax.ShapeDtypeStruct((_NC, n_nodes, D), jnp.float32),
        scratch_types=[
            pltpu.VMEM((2, C, D), jnp.float32),
            pltpu.VMEM((2, KSUB, CSUB), jnp.int32),
            pltpu.VMEM_SHARED((n_nodes, D), jnp.float32),
            pltpu.SemaphoreType.DMA,
            pltpu.SemaphoreType.DMA,
        ],
    )
    def seg_sum(rows_hbm, ids_hbm, zeros_hbm, out_hbm, rows_v, ids_v, acc, sem0, sem1):
        cid = lax.axis_index("c")
        sid = lax.axis_index("s")
        wid = cid * _NS + sid
        base = wid * rt
        sems = (sem0, sem1)

        def start(g, b):
            pltpu.async_copy(rows_hbm.at[pl.ds(base + g * C, C)], rows_v.at[b], sems[b])
            pltpu.async_copy(ids_hbm.at[wid * n_chunks + g], ids_v.at[b], sems[b])

        def wait(b):
            pltpu.make_async_copy(
                rows_hbm.at[pl.ds(0, C)], rows_v.at[b], sems[b]).wait()
            pltpu.make_async_copy(ids_hbm.at[0], ids_v.at[b], sems[b]).wait()

        def scatter(b):
            for j in range(KSUB):
                pltpu.sync_copy(
                    rows_v.at[b, pl.ds(j * CSUB, CSUB)],
                    acc.at[ids_v.at[b, j]],
                    add=True,
                )

        start(0, 0)

        @pl.when(sid < _NS - 1)
        def _():
            pltpu.sync_copy(
                zeros_hbm.at[pl.ds(sid * zr, zr)], acc.at[pl.ds(sid * zr, zr)])

        @pl.when(sid == _NS - 1)
        def _():
            pltpu.sync_copy(
                zeros_hbm.at[pl.ds((_NS - 1) * zr, zl)],
                acc.at[pl.ds((_NS - 1) * zr, zl)])

        plsc.subcore_barrier()

        def body(k, carry):
            g0 = k * 2
            wait(0)
            start(g0 + 1, 1)
            scatter(0)
            wait(1)
            start(g0 + 2, 0)
            scatter(1)
            return carry

        lax.fori_loop(0, (n_chunks - 1) // 2, body, 0)
        wait(0)
        scatter(0)
        plsc.subcore_barrier()

        @pl.when(sid < _NS - 1)
        def _():
            pltpu.sync_copy(
                acc.at[pl.ds(sid * zr, zr)], out_hbm.at[cid, pl.ds(sid * zr, zr)])

        @pl.when(sid == _NS - 1)
        def _():
            pltpu.sync_copy(
                acc.at[pl.ds((_NS - 1) * zr, zl)],
                out_hbm.at[cid, pl.ds((_NS - 1) * zr, zl)])

    zeros = jnp.zeros((n_nodes, D), jnp.float32)
    return seg_sum(feat_path, ids3d, zeros)


def _tc_dense(partials, feat_center, W_type, W_enc, W_dec, W_cls, b_cls):
    N, D = feat_center.shape
    P = W_type.shape[1]
    S = W_cls.shape[1]
    BN = 2000
    assert N % BN == 0

    def body(p_ref, fc_ref, wt_ref, we_ref, wd_ref, wc_ref, bc_ref, out1_ref, out2_ref):
        wcomb = jnp.dot(we_ref[...], wd_ref[...], preferred_element_type=jnp.float32)
        seg = p_ref[0] + p_ref[1]
        inj = jnp.dot(seg, wcomb, preferred_element_type=jnp.float32) * _W1
        center = jnp.dot(fc_ref[...], wt_ref[...], preferred_element_type=jnp.float32)
        logits = jnp.dot(center + inj, wc_ref[...],
                         preferred_element_type=jnp.float32) + bc_ref[...]
        m = jnp.max(logits, axis=1, keepdims=True)
        lse = jnp.log(jnp.sum(jnp.exp(logits - m), axis=1, keepdims=True)) + m
        out1_ref[...] = logits - lse
        out2_ref[...] = inj

    return pl.pallas_call(
        body,
        grid=(N // BN,),
        in_specs=[
            pl.BlockSpec((2, BN, D), lambda i: (0, i, 0)),
            pl.BlockSpec((BN, D), lambda i: (i, 0)),
            pl.BlockSpec((D, P), lambda i: (0, 0)),
            pl.BlockSpec((D, P), lambda i: (0, 0)),
            pl.BlockSpec((P, P), lambda i: (0, 0)),
            pl.BlockSpec((P, S), lambda i: (0, 0)),
            pl.BlockSpec((1, S), lambda i: (0, 0)),
        ],
        out_specs=[
            pl.BlockSpec((BN, S), lambda i: (i, 0)),
            pl.BlockSpec((BN, P), lambda i: (i, 0)),
        ],
        out_shape=[
            jax.ShapeDtypeStruct((N, S), jnp.float32),
            jax.ShapeDtypeStruct((N, P), jnp.float32),
        ],
    )(partials, feat_center, W_type, W_enc, W_dec, W_cls, b_cls.reshape(1, S))


def kernel(feat_center, feat_path, segment_ids, W_type, W_enc, W_dec, W_cls, b_cls):
    N = feat_center.shape[0]
    ids = segment_ids.astype(jnp.int32)
    partials = _sc_segment_sum(feat_path, ids, N)
    pre_embed, inj_scaled = _tc_dense(
        partials, feat_center, W_type, W_enc, W_dec, W_cls, b_cls)
    return (pre_embed, inj_scaled)


# speedup vs baseline: 7.7287x; 1.2463x over previous
import functools
import math

import jax
import jax.numpy as jnp
from jax import lax
from jax.experimental import pallas as pl
from jax.experimental.pallas import tpu as pltpu
from jax.experimental.pallas import tpu_sc as plsc

_NC = 2
_NS = 16
_NB = 4

_W1 = float(1.0 / (1.0 + math.exp(-0.5)))


def _sc_segment_sum(feat_path, ids, n_nodes):
    E, D = feat_path.shape
    NW = _NC * _NS
    rt = E // NW
    C = 80
    n_chunks = rt // C
    assert rt % C == 0 and C % 8 == 0 and n_chunks % _NB == 1 and n_chunks >= 9
    zr = (n_nodes // _NS) // 8 * 8
    zl = n_nodes - zr * (_NS - 1)

    ids3d = ids.reshape(E // C, 1, C)
    mesh = plsc.VectorSubcoreMesh(core_axis_name="c", subcore_axis_name="s")

    @functools.partial(
        pl.kernel,
        mesh=mesh,
        out_type=jax.ShapeDtypeStruct((_NC, n_nodes, D), jnp.float32),
        scratch_types=[
            pltpu.VMEM((_NB, C, D), jnp.float32),
            pltpu.VMEM((_NB, 1, C), jnp.int32),
            pltpu.VMEM_SHARED((n_nodes, D), jnp.float32),
            [pltpu.SemaphoreType.DMA] * _NB,
            [pltpu.SemaphoreType.DMA] * _NB,
        ],
    )
    def seg_sum(rows_hbm, ids_hbm, zeros_hbm, out_hbm, rows_v, ids_v, acc,
                sem_in, sem_sc):
        cid = lax.axis_index("c")
        sid = lax.axis_index("s")
        wid = cid * _NS + sid
        base = wid * rt

        def start_in(g, b):
            pltpu.async_copy(
                rows_hbm.at[pl.ds(base + g * C, C)], rows_v.at[b], sem_in[b])
            pltpu.async_copy(ids_hbm.at[wid * n_chunks + g], ids_v.at[b], sem_in[b])

        def wait_in(b):
            pltpu.make_async_copy(
                rows_hbm.at[pl.ds(0, C)], rows_v.at[b], sem_in[b]).wait()
            pltpu.make_async_copy(ids_hbm.at[0], ids_v.at[b], sem_in[b]).wait()

        def start_sc(b):
            pltpu.async_copy(
                rows_v.at[b], acc.at[ids_v.at[b, 0]], sem_sc[b], add=True)

        def wait_sc(b):
            pltpu.make_async_copy(
                rows_v.at[b], acc.at[ids_v.at[b, 0]], sem_sc[b]).wait()

        start_in(0, 0)
        start_in(1, 1)

        @pl.when(sid < _NS - 1)
        def _():
            pltpu.sync_copy(zeros_hbm.at[pl.ds(0, zr)], acc.at[pl.ds(sid * zr, zr)])

        @pl.when(sid == _NS - 1)
        def _():
            pltpu.sync_copy(zeros_hbm.at[pl.ds(0, zl)],
                            acc.at[pl.ds((_NS - 1) * zr, zl)])

        plsc.subcore_barrier()

        wait_in(0); start_sc(0); start_in(2, 2)
        wait_in(1); start_sc(1); start_in(3, 3)
        wait_in(2); start_sc(2); wait_sc(0); start_in(4, 0)
        wait_in(3); start_sc(3); wait_sc(1); start_in(5, 1)

        def body(k, carry):
            g = k * _NB
            wait_in(0); start_sc(0); wait_sc(2); start_in(g + 2, 2)
            wait_in(1); start_sc(1); wait_sc(3); start_in(g + 3, 3)
            wait_in(2); start_sc(2); wait_sc(0); start_in(g + 4, 0)
            wait_in(3); start_sc(3); wait_sc(1); start_in(g + 5, 1)
            return carry

        lax.fori_loop(1, (n_chunks - 1) // _NB - 1, body, 0)

        g = n_chunks - 5
        wait_in(0); start_sc(0); wait_sc(2); start_in(g + 2, 2)
        wait_in(1); start_sc(1); wait_sc(3); start_in(g + 3, 3)
        wait_in(2); start_sc(2); wait_sc(0); start_in(g + 4, 0)
        wait_in(3); start_sc(3); wait_sc(1)
        wait_in(0); start_sc(0); wait_sc(2)
        wait_sc(3); wait_sc(0)

        plsc.subcore_barrier()

        @pl.when(sid < _NS - 1)
        def _():
            pltpu.sync_copy(
                acc.at[pl.ds(sid * zr, zr)], out_hbm.at[cid, pl.ds(sid * zr, zr)])

        @pl.when(sid == _NS - 1)
        def _():
            pltpu.sync_copy(
                acc.at[pl.ds((_NS - 1) * zr, zl)],
                out_hbm.at[cid, pl.ds((_NS - 1) * zr, zl)])

    zeros = jnp.zeros((zl, D), jnp.float32)
    return seg_sum(feat_path, ids3d, zeros)


def _tc_dense(partials, feat_center, W_type, W_enc, W_dec, W_cls, b_cls):
    N, D = feat_center.shape
    P = W_type.shape[1]
    S = W_cls.shape[1]
    BN = 2000
    assert N % BN == 0

    def body(p_ref, fc_ref, wt_ref, we_ref, wd_ref, wc_ref, bc_ref, out1_ref, out2_ref):
        wcomb = jnp.dot(we_ref[...], wd_ref[...], preferred_element_type=jnp.float32)
        seg = p_ref[0] + p_ref[1]
        inj = jnp.dot(seg, wcomb, preferred_element_type=jnp.float32) * _W1
        center = jnp.dot(fc_ref[...], wt_ref[...], preferred_element_type=jnp.float32)
        logits = jnp.dot(center + inj, wc_ref[...],
                         preferred_element_type=jnp.float32) + bc_ref[...]
        m = jnp.max(logits, axis=1, keepdims=True)
        lse = jnp.log(jnp.sum(jnp.exp(logits - m), axis=1, keepdims=True)) + m
        out1_ref[...] = logits - lse
        out2_ref[...] = inj

    return pl.pallas_call(
        body,
        grid=(N // BN,),
        in_specs=[
            pl.BlockSpec((2, BN, D), lambda i: (0, i, 0)),
            pl.BlockSpec((BN, D), lambda i: (i, 0)),
            pl.BlockSpec((D, P), lambda i: (0, 0)),
            pl.BlockSpec((D, P), lambda i: (0, 0)),
            pl.BlockSpec((P, P), lambda i: (0, 0)),
            pl.BlockSpec((P, S), lambda i: (0, 0)),
            pl.BlockSpec((1, S), lambda i: (0, 0)),
        ],
        out_specs=[
            pl.BlockSpec((BN, S), lambda i: (i, 0)),
            pl.BlockSpec((BN, P), lambda i: (i, 0)),
        ],
        out_shape=[
            jax.ShapeDtypeStruct((N, S), jnp.float32),
            jax.ShapeDtypeStruct((N, P), jnp.float32),
        ],
    )(partials, feat_center, W_type, W_enc, W_dec, W_cls, b_cls.reshape(1, S))


def kernel(feat_center, feat_path, segment_ids, W_type, W_enc, W_dec, W_cls, b_cls):
    N = feat_center.shape[0]
    ids = segment_ids.astype(jnp.int32)
    partials = _sc_segment_sum(feat_path, ids, N)
    pre_embed, inj_scaled = _tc_dense(
        partials, feat_center, W_type, W_enc, W_dec, W_cls, b_cls)
    return (pre_embed, inj_scaled)
